# Initial kernel scaffold; baseline (speedup 1.0000x reference)
import jax
import jax.numpy as jnp
from jax.experimental import pallas as pl


def kernel(x, edge_index, edge_attr, positions, batch, emb2d, emb3d, We, be, W1, b1, W2, b2, g2d, bt2d, Wf1, bf1, Wf2, bf2, Wpre, Wpost, bpost, Wout, bout, g3d, bt3d, Wm1, bm1, gm1, btm1, Wm2, bm2, gm2, btm2):
    raise NotImplementedError("write your pallas kernel here")


# speedup vs baseline: 2.1347x; 2.1347x over previous
import functools

import jax
import jax.numpy as jnp
from jax import lax
from jax.experimental import pallas as pl
from jax.experimental.pallas import tpu as pltpu
from jax.experimental.pallas import tpu_sc as plsc

N = 10000
NPAD = 10240
E = 320000
EPAD = 323584
D = 16
H = 128
NB = 2
NG = 64
NGAUSS = 50
CUTOFF = 10.0
NUM_CLASS = 119

CHUNK = 128
NSC = 2
NTILE = 16
NW = NSC * NTILE
GEOM_EPW = EPAD // NW
GEOM_NCHUNK = GEOM_EPW // CHUNK
EPT = EPAD // NTILE
PASS_NCHUNK = EPT // CHUNK
NROWS_PT = NPAD // NTILE

@functools.lru_cache(maxsize=None)
def _sc_mesh():
    return plsc.VectorSubcoreMesh(
        core_axis_name="c", subcore_axis_name="s",
        num_cores=NSC, num_subcores=NTILE)


def _geom_body(posf, src, dst, out, posv, sidx, didx, d2b):
    cid = lax.axis_index("c")
    sid = lax.axis_index("s")
    wid = sid * NSC + cid
    pltpu.sync_copy(posf, posv)
    base0 = wid * GEOM_EPW

    def chunk(k, carry):
        b = base0 + k * CHUNK
        pltpu.sync_copy(src.at[pl.ds(b, CHUNK)], sidx)
        pltpu.sync_copy(dst.at[pl.ds(b, CHUNK)], didx)
        for j in range(CHUNK // 16):
            sl = pl.ds(j * 16, 16)
            si3 = sidx[sl] * 3
            di3 = didx[sl] * 3
            dx = plsc.load_gather(posv, [si3]) - plsc.load_gather(posv, [di3])
            dy = plsc.load_gather(posv, [si3 + 1]) - plsc.load_gather(posv, [di3 + 1])
            dz = plsc.load_gather(posv, [si3 + 2]) - plsc.load_gather(posv, [di3 + 2])
            d2b[sl] = dx * dx + dy * dy + dz * dz
        pltpu.sync_copy(d2b, out.at[pl.ds(b, CHUNK)])
        return carry

    lax.fori_loop(0, GEOM_NCHUNK, chunk, 0)


@functools.lru_cache(maxsize=None)
def _geom_kernel():
    return pl.kernel(
        _geom_body,
        out_type=jax.ShapeDtypeStruct((EPAD,), jnp.float32),
        mesh=_sc_mesh(),
        compiler_params=pltpu.CompilerParams(needs_layout_passes=False),
        scratch_types=[
            pltpu.VMEM((3 * NPAD,), jnp.float32),
            pltpu.VMEM((CHUNK,), jnp.int32),
            pltpu.VMEM((CHUNK,), jnp.int32),
            pltpu.VMEM((CHUNK,), jnp.float32),
        ],
    )


def _geom_call(*args):
    return _geom_kernel()(*args)


def _pass_body(x2, e, hp, wf, src, dst, zer, out2, out3, gix, six, pb, gb, sem, acc):
    cid = lax.axis_index("c")
    sid = lax.axis_index("s")
    base0 = sid * EPT
    rs = pl.ds(sid * NROWS_PT, NROWS_PT)

    def run(tbl, pay, gs, ss, outp, mul):
        pltpu.sync_copy(zer, acc.at[rs])
        plsc.subcore_barrier()

        def chunk(k, carry):
            b = base0 + k * CHUNK
            pltpu.sync_copy(gs.at[pl.ds(b, CHUNK)], gix)
            pltpu.sync_copy(ss.at[pl.ds(b, CHUNK)], six)
            pltpu.sync_copy(pay.at[pl.ds(b, CHUNK)], pb)
            pltpu.async_copy(tbl.at[gix], gb, sem).wait()

            def row(r, rc):
                for j in range(H // 16):
                    sl = pl.ds(j * 16, 16)
                    a = gb[r, sl]
                    p = pb[r, sl]
                    if mul:
                        gb[r, sl] = a * p
                    else:
                        gb[r, sl] = jnp.maximum(a + p, 0.0)
                return rc

            lax.fori_loop(0, CHUNK, row, 0)
            pltpu.sync_copy(gb, acc.at[six], add=True)
            return carry

        lax.fori_loop(0, PASS_NCHUNK, chunk, 0)
        plsc.subcore_barrier()
        pltpu.sync_copy(acc.at[rs], outp.at[rs])

    @pl.when(cid == 0)
    def _():
        run(x2, e, src, dst, out2, False)

    @pl.when(cid == 1)
    def _():
        run(hp, wf, dst, src, out3, True)


@functools.lru_cache(maxsize=None)
def _pass_kernel():
    return pl.kernel(
        _pass_body,
        out_type=(
            jax.ShapeDtypeStruct((NPAD, H), jnp.float32),
            jax.ShapeDtypeStruct((NPAD, H), jnp.float32),
        ),
        mesh=_sc_mesh(),
        compiler_params=pltpu.CompilerParams(needs_layout_passes=False),
        scratch_types=[
            pltpu.VMEM((CHUNK,), jnp.int32),
            pltpu.VMEM((CHUNK,), jnp.int32),
            pltpu.VMEM((CHUNK, H), jnp.float32),
            pltpu.VMEM((CHUNK, H), jnp.float32),
            pltpu.SemaphoreType.DMA,
            pltpu.VMEM_SHARED((NPAD, H), jnp.float32),
        ],
    )


def _pass_call(*args):
    return _pass_kernel()(*args)


def _bn_masked(h, g, b):
    mask = lax.broadcasted_iota(jnp.int32, (NPAD, 1), 0) < N
    hm = jnp.where(mask, h, 0.0)
    mu = jnp.sum(hm, axis=0, keepdims=True) * (1.0 / N)
    xc = h - mu
    var = jnp.sum(jnp.where(mask, xc * xc, 0.0), axis=0, keepdims=True) * (1.0 / N)
    return g * xc * lax.rsqrt(var + 1e-5) + b


def _bn_rows(h, g, b, rows):
    mu = jnp.mean(h, axis=0, keepdims=True)
    xc = h - mu
    var = jnp.mean(xc * xc, axis=0, keepdims=True)
    return g * xc * lax.rsqrt(var + 1e-5) + b


def _ssp_tc(v):
    return jnp.maximum(v, 0.0) + jnp.log1p(jnp.exp(-jnp.abs(v))) - 0.6931471805599453


def _embed_body(xi, e2, e3, wpre, x2o, x3o, hpo):
    ids = xi[...]
    lanes = lax.broadcasted_iota(jnp.int32, (NPAD, H), 1)
    oh = (ids == lanes).astype(jnp.float32)
    x2 = jnp.dot(oh, e2[...], preferred_element_type=jnp.float32,
                 precision=lax.Precision.HIGHEST)
    x3 = jnp.dot(oh, e3[...], preferred_element_type=jnp.float32,
                 precision=lax.Precision.HIGHEST)
    x2o[...] = x2
    x3o[...] = x3
    hpo[...] = jnp.dot(x3, wpre[...], preferred_element_type=jnp.float32)


def _embed_call(xi, e2, e3, wpre):
    return pl.pallas_call(
        _embed_body,
        out_shape=(
            jax.ShapeDtypeStruct((NPAD, H), jnp.float32),
            jax.ShapeDtypeStruct((NPAD, H), jnp.float32),
            jax.ShapeDtypeStruct((NPAD, H), jnp.float32),
        ),
    )(xi, e2, e3, wpre)


_TE = 2048
_EGRID = EPAD // _TE


def _edge_body(d2r, ear, Wer, ber, Wf1r, bf1r, Wf2r, bf2r, e0o, e1o, w0o, w1o):
    d2 = d2r[...]
    d = jnp.sqrt(d2 + 1e-12)
    step = CUTOFF / (NGAUSS - 1)
    offs = lax.broadcasted_iota(jnp.int32, (1, NGAUSS), 1).astype(jnp.float32) * step
    coeff = -0.5 / (step * step)
    rbf = jnp.exp(coeff * (d - offs) ** 2)
    C = 0.5 * (jnp.cos(d * (jnp.pi / CUTOFF)) + 1.0)
    C = C * (d < CUTOFF).astype(jnp.float32)
    ea = ear[...]
    for i, (eo, wo) in enumerate(((e0o, w0o), (e1o, w1o))):
        ei = jnp.dot(ea, Wer[i], preferred_element_type=jnp.float32) + ber[i]
        eo[...] = jnp.maximum(ei, 0.0)
        u = _ssp_tc(jnp.dot(rbf, Wf1r[i], preferred_element_type=jnp.float32) + bf1r[i])
        wf = jnp.dot(u, Wf2r[i], preferred_element_type=jnp.float32) + bf2r[i]
        wo[...] = wf * C


def _edge_call(d2, ea, We, be, Wf1, bf1, Wf2, bf2):
    full = lambda *s: pl.BlockSpec(s, lambda i: (0,) * len(s))
    return pl.pallas_call(
        _edge_body,
        grid=(_EGRID,),
        in_specs=[
            pl.BlockSpec((_TE, 1), lambda i: (i, 0)),
            pl.BlockSpec((_TE, D), lambda i: (i, 0)),
            full(NB, D, H),
            full(NB, H),
            full(NB, NGAUSS, H),
            full(NB, H),
            full(NB, H, H),
            full(NB, H),
        ],
        out_specs=[pl.BlockSpec((_TE, H), lambda i: (i, 0))] * 4,
        out_shape=[jax.ShapeDtypeStruct((EPAD, H), jnp.float32)] * 4,
    )(d2, ea, We, be, Wf1, bf1, Wf2, bf2)


def _n2d_body(x2r, aggr, W1r, b1r, W2r, b2r, gr, btr, outr):
    x2 = x2r[...]
    h = x2 + aggr[...]
    t = jnp.maximum(jnp.dot(h, W1r[...], preferred_element_type=jnp.float32) + b1r[...], 0.0)
    h2 = jnp.dot(t, W2r[...], preferred_element_type=jnp.float32) + b2r[...]
    outr[...] = x2 + _bn_masked(h2, gr[...], btr[...])


def _n2d_call(x2, agg, W1, b1, W2, b2, g, bt):
    return pl.pallas_call(
        _n2d_body,
        out_shape=jax.ShapeDtypeStruct((NPAD, H), jnp.float32),
    )(x2, agg, W1, b1, W2, b2, g, bt)


def _n3d_body(x3r, a3r, Wpostr, bpostr, Woutr, boutr, gr, btr, Wprer, x3o, hpo):
    x3 = x3r[...]
    t = _ssp_tc(jnp.dot(a3r[...], Wpostr[...], preferred_element_type=jnp.float32) + bpostr[...])
    h3 = jnp.dot(t, Woutr[...], preferred_element_type=jnp.float32) + boutr[...]
    x3n = x3 + _bn_masked(h3, gr[...], btr[...])
    x3o[...] = x3n
    hpo[...] = jnp.dot(x3n, Wprer[...], preferred_element_type=jnp.float32)


def _n3d_call(x3, a3, Wpost, bpost, Wout, bout, g, bt, Wpre_next):
    return pl.pallas_call(
        _n3d_body,
        out_shape=(
            jax.ShapeDtypeStruct((NPAD, H), jnp.float32),
            jax.ShapeDtypeStruct((NPAD, H), jnp.float32),
        ),
    )(x3, a3, Wpost, bpost, Wout, bout, g, bt, Wpre_next)


def _final_body(x2r, x3r, btr, Wm1r, bm1r, gm1r, btm1r, Wm2r, bm2r, gm2r, btm2r, zo):
    bids = btr[...]
    segs = lax.broadcasted_iota(jnp.int32, (NG, NPAD), 0)
    valid = lax.broadcasted_iota(jnp.int32, (1, NPAD), 1) < N
    oh = jnp.where((bids == segs) & valid, 1.0, 0.0)
    cnt = jnp.maximum(jnp.sum(oh, axis=1, keepdims=True), 1.0)
    rep2 = jnp.dot(oh, x2r[...], preferred_element_type=jnp.float32,
                   precision=lax.Precision.HIGHEST) / cnt
    rep3 = jnp.dot(oh, x3r[...], preferred_element_type=jnp.float32,
                   precision=lax.Precision.HIGHEST) / cnt
    z = jnp.concatenate([rep2, rep3], axis=1)
    u = jnp.dot(z, Wm1r[...], preferred_element_type=jnp.float32) + bm1r[...]
    u = jnp.maximum(_bn_rows(u, gm1r[...], btm1r[...], NG), 0.0)
    u = jnp.dot(u, Wm2r[...], preferred_element_type=jnp.float32) + bm2r[...]
    zo[...] = jnp.maximum(_bn_rows(u, gm2r[...], btm2r[...], NG), 0.0)


def _final_call(x2, x3, bt, Wm1, bm1, gm1, btm1, Wm2, bm2, gm2, btm2):
    return pl.pallas_call(
        _final_body,
        out_shape=jax.ShapeDtypeStruct((NG, 2 * H), jnp.float32),
    )(x2, x3, bt, Wm1, bm1, gm1, btm1, Wm2, bm2, gm2, btm2)


def kernel(x, edge_index, edge_attr, positions, batch, emb2d, emb3d, We, be, W1, b1, W2, b2, g2d, bt2d, Wf1, bf1, Wf2, bf2, Wpre, Wpost, bpost, Wout, bout, g3d, bt3d, Wm1, bm1, gm1, btm1, Wm2, bm2, gm2, btm2):
    src = edge_index[0].astype(jnp.int32)
    dst = edge_index[1].astype(jnp.int32)
    pad_e = EPAD - E
    pad_idx = N + (jnp.arange(pad_e, dtype=jnp.int32) % (NPAD - N))
    src_p = jnp.concatenate([src, pad_idx])
    dst_p = jnp.concatenate([dst, pad_idx])
    ea_p = jnp.pad(edge_attr, ((0, pad_e), (0, 0)))
    posf = jnp.pad(positions, ((0, NPAD - N), (0, 0))).reshape(-1)
    xi = jnp.pad(x.astype(jnp.int32), (0, NPAD - N)).reshape(NPAD, 1)
    bt_p = jnp.pad(batch.astype(jnp.int32), (0, NPAD - N), constant_values=NG).reshape(1, NPAD)
    e2p = jnp.pad(emb2d, ((0, H - NUM_CLASS), (0, 0)))
    e3p = jnp.pad(emb3d, ((0, H - NUM_CLASS), (0, 0)))
    zer = jnp.zeros((NROWS_PT, H), jnp.float32)

    d2 = _geom_call(posf, src_p, dst_p)
    x2, x3, hp = _embed_call(xi, e2p, e3p, Wpre[0])
    e0, e1, w0, w1 = _edge_call(
        d2.reshape(EPAD, 1), ea_p, We, be, Wf1, bf1, Wf2, bf2)
    evs = (e0, e1)
    wvs = (w0, w1)
    for i in range(NB):
        agg2, agg3 = _pass_call(x2, evs[i], hp, wvs[i], src_p, dst_p, zer)
        x2 = _n2d_call(x2, agg2, W1[i], b1[i], W2[i], b2[i], g2d[i], bt2d[i])
        x3, hp = _n3d_call(x3, agg3, Wpost[i], bpost[i], Wout[i], bout[i],
                           g3d[i], bt3d[i], Wpre[(i + 1) % NB])
    return _final_call(x2, x3, bt_p, Wm1, bm1, gm1, btm1, Wm2, bm2, gm2, btm2)


# speedup vs baseline: 2.7763x; 1.3005x over previous
import functools

import jax
import jax.numpy as jnp
from jax import lax
from jax.experimental import pallas as pl
from jax.experimental.pallas import tpu as pltpu
from jax.experimental.pallas import tpu_sc as plsc

N = 10000
NPAD = 10112
E = 320000
EPAD = 323584
D = 16
H = 128
NB = 2
NG = 64
NGAUSS = 50
CUTOFF = 10.0
NUM_CLASS = 119

CHUNK = 128
PCHUNK = 64
NSC = 2
NTILE = 16
NW = NSC * NTILE
GEOM_EPW = EPAD // NW
GEOM_NCHUNK = GEOM_EPW // CHUNK
EPT = EPAD // NTILE
PASS_NCHUNK = EPT // PCHUNK
NROWS_PT = NPAD // NTILE

@functools.lru_cache(maxsize=None)
def _sc_mesh():
    return plsc.VectorSubcoreMesh(
        core_axis_name="c", subcore_axis_name="s",
        num_cores=NSC, num_subcores=NTILE)


def _geom_body(posf, sd, out, posv, ib, ob, is0, is1, os0, os1):
    cid = lax.axis_index("c")
    sid = lax.axis_index("s")
    wid = sid * NSC + cid
    pltpu.sync_copy(posf, posv)
    base0 = wid * GEOM_EPW
    isem = (is0, is1)
    osem = (os0, os1)

    def loads(k, b):
        off = base0 + k * CHUNK
        pltpu.async_copy(sd.at[:, pl.ds(off, CHUNK)], ib.at[b], isem[b])

    def wait_idx(b):
        pltpu.make_async_copy(sd.at[:, pl.ds(0, CHUNK)], ib.at[b], isem[b]).wait()

    def wait_out(b):
        pltpu.make_async_copy(ob.at[b], out.at[pl.ds(0, CHUNK)], osem[b]).wait()

    def compute(k, b):
        for j in range(CHUNK // 16):
            sl = pl.ds(j * 16, 16)
            si3 = ib[b, 0, sl] * 3
            di3 = ib[b, 1, sl] * 3
            dx = plsc.load_gather(posv, [si3]) - plsc.load_gather(posv, [di3])
            dy = plsc.load_gather(posv, [si3 + 1]) - plsc.load_gather(posv, [di3 + 1])
            dz = plsc.load_gather(posv, [si3 + 2]) - plsc.load_gather(posv, [di3 + 2])
            ob[b, sl] = dx * dx + dy * dy + dz * dz
        pltpu.async_copy(ob.at[b], out.at[pl.ds(base0 + k * CHUNK, CHUNK)], osem[b])

    loads(0, 0)

    def pair(p, carry):
        kk = p * 2
        for b in (0, 1):
            k = kk + b

            @pl.when(k + 1 < GEOM_NCHUNK)
            def _():
                loads(k + 1, 1 - b)

            wait_idx(b)

            @pl.when(k >= 2)
            def _():
                wait_out(b)

            compute(k, b)
        return carry

    lax.fori_loop(0, (GEOM_NCHUNK - 1) // 2, pair, 0)
    k_last = GEOM_NCHUNK - 1
    wait_idx(0)
    wait_out(0)
    compute(k_last, 0)
    wait_out(1)
    wait_out(0)


@functools.lru_cache(maxsize=None)
def _geom_kernel():
    return pl.kernel(
        _geom_body,
        out_type=jax.ShapeDtypeStruct((EPAD,), jnp.float32),
        mesh=_sc_mesh(),
        compiler_params=pltpu.CompilerParams(needs_layout_passes=False),
        scratch_types=[
            pltpu.VMEM((3 * NPAD,), jnp.float32),
            pltpu.VMEM((2, 2, CHUNK), jnp.int32),
            pltpu.VMEM((2, CHUNK), jnp.float32),
            pltpu.SemaphoreType.DMA,
            pltpu.SemaphoreType.DMA,
            pltpu.SemaphoreType.DMA,
            pltpu.SemaphoreType.DMA,
        ],
    )


def _geom_call(*args):
    return _geom_kernel()(*args)


def _pass_body(x2, e, hp, wf, sdi, zer, out2, out3,
               ib, sib, pb, gb, sb,
               is0, is1, ps0, ps1, gs0, gs1, ss0, ss1, acc):
    cid = lax.axis_index("c")
    sid = lax.axis_index("s")
    base0 = sid * EPT
    rs = pl.ds(sid * NROWS_PT, NROWS_PT)
    isem = (is0, is1)
    psem = (ps0, ps1)
    gsem = (gs0, gs1)
    ssem = (ss0, ss1)

    def run(tbl, pay, gI, sI, outp, mul):
        pltpu.sync_copy(zer, acc.at[rs])
        plsc.subcore_barrier()

        def loads(k, b):
            off = base0 + k * PCHUNK
            pltpu.async_copy(
                sdi.at[pl.ds(base0 * 2 + k * (2 * PCHUNK), 2 * PCHUNK)],
                ib.at[b], isem[b])
            pltpu.async_copy(pay.at[pl.ds(off, PCHUNK)], pb.at[b], psem[b])

        def wait_idx(b):
            pltpu.make_async_copy(
                sdi.at[pl.ds(0, 2 * PCHUNK)], ib.at[b], isem[b]).wait()

        def wait_pay(b):
            pltpu.make_async_copy(pay.at[pl.ds(0, PCHUNK)], pb.at[b], psem[b]).wait()

        def start_gather(b):
            pltpu.async_copy(
                tbl.at[ib.at[b, pl.ds(gI * PCHUNK, PCHUNK)]], gb.at[b], gsem[b])

        def wait_gather(b):
            pltpu.make_async_copy(
                tbl.at[ib.at[b, pl.ds(gI * PCHUNK, PCHUNK)]], gb.at[b],
                gsem[b]).wait()

        def start_scatter(b):
            pltpu.async_copy(sb.at[b], acc.at[sib.at[b]], ssem[b], add=True)

        def wait_scatter(b):
            pltpu.make_async_copy(sb.at[b], acc.at[sib.at[b]], ssem[b]).wait()

        def compute(b):
            for j in range(PCHUNK // 16):
                sl = pl.ds(j * 16, 16)
                sib[b, sl] = ib[b, pl.ds(sI * PCHUNK + j * 16, 16)]

            def row(r, rc):
                for j in range(H // 16):
                    sl = pl.ds(j * 16, 16)
                    a = gb[b, r, sl]
                    p = pb[b, r, sl]
                    if mul:
                        sb[b, r, sl] = a * p
                    else:
                        sb[b, r, sl] = jnp.maximum(a + p, 0.0)
                return rc

            lax.fori_loop(0, PCHUNK, row, 0)

        loads(0, 0)
        loads(1, 1)
        wait_idx(0)
        start_gather(0)

        def pair(p, carry):
            kk = p * 2
            for b in (0, 1):
                k = kk + b

                @pl.when(k + 1 < PASS_NCHUNK)
                def _():
                    wait_idx(1 - b)
                    start_gather(1 - b)

                wait_gather(b)
                wait_pay(b)

                @pl.when(k >= 2)
                def _():
                    wait_scatter(b)

                compute(b)
                start_scatter(b)

                @pl.when(k + 2 < PASS_NCHUNK)
                def _():
                    loads(k + 2, b)
            return carry

        lax.fori_loop(0, PASS_NCHUNK // 2, pair, 0)
        wait_scatter(0)
        wait_scatter(1)
        plsc.subcore_barrier()
        pltpu.sync_copy(acc.at[rs], outp.at[rs])

    @pl.when(cid == 0)
    def _():
        run(x2, e, 0, 1, out2, False)

    @pl.when(cid == 1)
    def _():
        run(hp, wf, 1, 0, out3, True)


@functools.lru_cache(maxsize=None)
def _pass_kernel():
    return pl.kernel(
        _pass_body,
        out_type=(
            jax.ShapeDtypeStruct((NPAD, H), jnp.float32),
            jax.ShapeDtypeStruct((NPAD, H), jnp.float32),
        ),
        mesh=_sc_mesh(),
        compiler_params=pltpu.CompilerParams(needs_layout_passes=False),
        scratch_types=[
            pltpu.VMEM((2, 2 * PCHUNK), jnp.int32),
            pltpu.VMEM((2, PCHUNK), jnp.int32),
            pltpu.VMEM((2, PCHUNK, H), jnp.float32),
            pltpu.VMEM((2, PCHUNK, H), jnp.float32),
            pltpu.VMEM((2, PCHUNK, H), jnp.float32),
            pltpu.SemaphoreType.DMA,
            pltpu.SemaphoreType.DMA,
            pltpu.SemaphoreType.DMA,
            pltpu.SemaphoreType.DMA,
            pltpu.SemaphoreType.DMA,
            pltpu.SemaphoreType.DMA,
            pltpu.SemaphoreType.DMA,
            pltpu.SemaphoreType.DMA,
            pltpu.VMEM_SHARED((NPAD, H), jnp.float32),
        ],
    )


def _pass_call(*args):
    return _pass_kernel()(*args)


def _bn_masked(h, g, b):
    mask = lax.broadcasted_iota(jnp.int32, (NPAD, 1), 0) < N
    hm = jnp.where(mask, h, 0.0)
    mu = jnp.sum(hm, axis=0, keepdims=True) * (1.0 / N)
    xc = h - mu
    var = jnp.sum(jnp.where(mask, xc * xc, 0.0), axis=0, keepdims=True) * (1.0 / N)
    return g * xc * lax.rsqrt(var + 1e-5) + b


def _bn_rows(h, g, b, rows):
    mu = jnp.mean(h, axis=0, keepdims=True)
    xc = h - mu
    var = jnp.mean(xc * xc, axis=0, keepdims=True)
    return g * xc * lax.rsqrt(var + 1e-5) + b


def _ssp_tc(v):
    return jnp.maximum(v, 0.0) + jnp.log1p(jnp.exp(-jnp.abs(v))) - 0.6931471805599453


def _embed_body(xi, e2, e3, wpre, x2o, x3o, hpo):
    ids = xi[...]
    lanes = lax.broadcasted_iota(jnp.int32, (NPAD, H), 1)
    oh = (ids == lanes).astype(jnp.float32)
    x2 = jnp.dot(oh, e2[...], preferred_element_type=jnp.float32,
                 precision=lax.Precision.HIGHEST)
    x3 = jnp.dot(oh, e3[...], preferred_element_type=jnp.float32,
                 precision=lax.Precision.HIGHEST)
    x2o[...] = x2
    x3o[...] = x3
    hpo[...] = jnp.dot(x3, wpre[...], preferred_element_type=jnp.float32)


def _embed_call(xi, e2, e3, wpre):
    return pl.pallas_call(
        _embed_body,
        out_shape=(
            jax.ShapeDtypeStruct((NPAD, H), jnp.float32),
            jax.ShapeDtypeStruct((NPAD, H), jnp.float32),
            jax.ShapeDtypeStruct((NPAD, H), jnp.float32),
        ),
    )(xi, e2, e3, wpre)


_TE = 2048
_EGRID = EPAD // _TE


def _edge_body(d2r, ear, Wer, ber, Wf1r, bf1r, Wf2r, bf2r, eo, wo):
    d2 = d2r[...]
    d = jnp.sqrt(d2 + 1e-12)
    step = CUTOFF / (NGAUSS - 1)
    offs = lax.broadcasted_iota(jnp.int32, (1, NGAUSS), 1).astype(jnp.float32) * step
    coeff = -0.5 / (step * step)
    rbf = jnp.exp(coeff * (d - offs) ** 2)
    C = 0.5 * (jnp.cos(d * (jnp.pi / CUTOFF)) + 1.0)
    C = C * (d < CUTOFF).astype(jnp.float32)
    ea = ear[...]
    ei = jnp.dot(ea, Wer[...], preferred_element_type=jnp.float32) + ber[...]
    eo[...] = jnp.maximum(ei, 0.0)
    u = _ssp_tc(jnp.dot(rbf, Wf1r[...], preferred_element_type=jnp.float32) + bf1r[...])
    wf = jnp.dot(u, Wf2r[...], preferred_element_type=jnp.float32) + bf2r[...]
    wo[...] = wf * C


def _edge_call(d2, ea, We_i, be_i, Wf1_i, bf1_i, Wf2_i, bf2_i):
    full = lambda *s: pl.BlockSpec(s, lambda i: (0,) * len(s))
    return pl.pallas_call(
        _edge_body,
        grid=(_EGRID,),
        in_specs=[
            pl.BlockSpec((_TE, 1), lambda i: (i, 0)),
            pl.BlockSpec((_TE, D), lambda i: (i, 0)),
            full(D, H),
            full(H),
            full(NGAUSS, H),
            full(H),
            full(H, H),
            full(H),
        ],
        out_specs=[pl.BlockSpec((_TE, H), lambda i: (i, 0))] * 2,
        out_shape=[jax.ShapeDtypeStruct((EPAD, H), jnp.float32)] * 2,
    )(d2, ea, We_i, be_i, Wf1_i, bf1_i, Wf2_i, bf2_i)


def _n2d_body(x2r, aggr, W1r, b1r, W2r, b2r, gr, btr, outr):
    x2 = x2r[...]
    h = x2 + aggr[...]
    t = jnp.maximum(jnp.dot(h, W1r[...], preferred_element_type=jnp.float32) + b1r[...], 0.0)
    h2 = jnp.dot(t, W2r[...], preferred_element_type=jnp.float32) + b2r[...]
    outr[...] = x2 + _bn_masked(h2, gr[...], btr[...])


def _n2d_call(x2, agg, W1, b1, W2, b2, g, bt):
    return pl.pallas_call(
        _n2d_body,
        out_shape=jax.ShapeDtypeStruct((NPAD, H), jnp.float32),
    )(x2, agg, W1, b1, W2, b2, g, bt)


def _n3d_body(x3r, a3r, Wpostr, bpostr, Woutr, boutr, gr, btr, Wprer, x3o, hpo):
    x3 = x3r[...]
    t = _ssp_tc(jnp.dot(a3r[...], Wpostr[...], preferred_element_type=jnp.float32) + bpostr[...])
    h3 = jnp.dot(t, Woutr[...], preferred_element_type=jnp.float32) + boutr[...]
    x3n = x3 + _bn_masked(h3, gr[...], btr[...])
    x3o[...] = x3n
    hpo[...] = jnp.dot(x3n, Wprer[...], preferred_element_type=jnp.float32)


def _n3d_call(x3, a3, Wpost, bpost, Wout, bout, g, bt, Wpre_next):
    return pl.pallas_call(
        _n3d_body,
        out_shape=(
            jax.ShapeDtypeStruct((NPAD, H), jnp.float32),
            jax.ShapeDtypeStruct((NPAD, H), jnp.float32),
        ),
    )(x3, a3, Wpost, bpost, Wout, bout, g, bt, Wpre_next)


def _final_body(x2r, x3r, btr, Wm1r, bm1r, gm1r, btm1r, Wm2r, bm2r, gm2r, btm2r, zo):
    bids = btr[...]
    segs = lax.broadcasted_iota(jnp.int32, (NG, NPAD), 0)
    valid = lax.broadcasted_iota(jnp.int32, (1, NPAD), 1) < N
    oh = jnp.where((bids == segs) & valid, 1.0, 0.0)
    cnt = jnp.maximum(jnp.sum(oh, axis=1, keepdims=True), 1.0)
    rep2 = jnp.dot(oh, x2r[...], preferred_element_type=jnp.float32,
                   precision=lax.Precision.HIGHEST) / cnt
    rep3 = jnp.dot(oh, x3r[...], preferred_element_type=jnp.float32,
                   precision=lax.Precision.HIGHEST) / cnt
    z = jnp.concatenate([rep2, rep3], axis=1)
    u = jnp.dot(z, Wm1r[...], preferred_element_type=jnp.float32) + bm1r[...]
    u = jnp.maximum(_bn_rows(u, gm1r[...], btm1r[...], NG), 0.0)
    u = jnp.dot(u, Wm2r[...], preferred_element_type=jnp.float32) + bm2r[...]
    zo[...] = jnp.maximum(_bn_rows(u, gm2r[...], btm2r[...], NG), 0.0)


def _final_call(x2, x3, bt, Wm1, bm1, gm1, btm1, Wm2, bm2, gm2, btm2):
    return pl.pallas_call(
        _final_body,
        out_shape=jax.ShapeDtypeStruct((NG, 2 * H), jnp.float32),
    )(x2, x3, bt, Wm1, bm1, gm1, btm1, Wm2, bm2, gm2, btm2)


def kernel(x, edge_index, edge_attr, positions, batch, emb2d, emb3d, We, be, W1, b1, W2, b2, g2d, bt2d, Wf1, bf1, Wf2, bf2, Wpre, Wpost, bpost, Wout, bout, g3d, bt3d, Wm1, bm1, gm1, btm1, Wm2, bm2, gm2, btm2):
    src = edge_index[0].astype(jnp.int32)
    dst = edge_index[1].astype(jnp.int32)
    pad_e = EPAD - E
    pad_idx = N + (jnp.arange(pad_e, dtype=jnp.int32) % (NPAD - N))
    src_p = jnp.concatenate([src, pad_idx])
    dst_p = jnp.concatenate([dst, pad_idx])
    sd = jnp.stack([src_p, dst_p])
    sdi = jnp.stack([src_p.reshape(-1, PCHUNK), dst_p.reshape(-1, PCHUNK)],
                    axis=1).reshape(-1)
    ea_p = jnp.pad(edge_attr, ((0, pad_e), (0, 0)))
    posf = jnp.pad(positions, ((0, NPAD - N), (0, 0))).reshape(-1)
    xi = jnp.pad(x.astype(jnp.int32), (0, NPAD - N)).reshape(NPAD, 1)
    bt_p = jnp.pad(batch.astype(jnp.int32), (0, NPAD - N), constant_values=NG).reshape(1, NPAD)
    e2p = jnp.pad(emb2d, ((0, H - NUM_CLASS), (0, 0)))
    e3p = jnp.pad(emb3d, ((0, H - NUM_CLASS), (0, 0)))
    zer = jnp.zeros((NROWS_PT, H), jnp.float32)

    d2 = _geom_call(posf, sd)
    x2, x3, hp = _embed_call(xi, e2p, e3p, Wpre[0])
    d2r = d2.reshape(EPAD, 1)
    for i in range(NB):
        e_i, w_i = _edge_call(d2r, ea_p, We[i], be[i], Wf1[i], bf1[i],
                              Wf2[i], bf2[i])
        agg2, agg3 = _pass_call(x2, e_i, hp, w_i, sdi, zer)
        x2 = _n2d_call(x2, agg2, W1[i], b1[i], W2[i], b2[i], g2d[i], bt2d[i])
        x3, hp = _n3d_call(x3, agg3, Wpost[i], bpost[i], Wout[i], bout[i],
                           g3d[i], bt3d[i], Wpre[(i + 1) % NB])
    return _final_call(x2, x3, bt_p, Wm1, bm1, gm1, btm1, Wm2, bm2, gm2, btm2)
